# baseline (speedup 1.0000x reference)
import functools

import jax
import jax.numpy as jnp
from jax import lax
from jax.experimental import pallas as pl
from jax.experimental.pallas import tpu as pltpu
from jax.experimental.pallas import tpu_sc as plsc

N_BINS = 15
N_ROWS = 16384
N_COLS = 1000
NTILES = 32
ROWS_PER_TILE = N_ROWS // NTILES
CH = 32
NCHUNKS = ROWS_PER_TILE // CH
NVREG = N_COLS // 16
TAIL = N_COLS - NVREG * 16
NEG = -1e30


def _row_body(buf, lab_ref, rb, lo16, hi16, lane):
    def allreduce(v, op):
        for k in (8, 4, 2, 1):
            rb[...] = v
            vs = plsc.load_gather(rb, [jnp.bitwise_xor(lane, k)])
            v = op(v, vs)
        return v

    def body(r, carry):
        cnt, csum, asum = carry
        base = r * N_COLS
        accm = jnp.full((16,), NEG, jnp.float32)
        accs = jnp.zeros((16,), jnp.float32)
        for i in range(NVREG):
            v = buf[pl.ds(base + i * 16, 16)]
            accm = jnp.maximum(accm, v)
            accs = accs + jnp.exp(v)
        vt = buf[pl.ds(base + N_COLS - 16, 16)]
        sel = lane >= (16 - TAIL)
        accm = jnp.maximum(accm, jnp.where(sel, vt, NEG))
        accs = accs + jnp.where(sel, jnp.exp(vt), 0.0)

        m16 = allreduce(accm, jnp.maximum)
        s16 = allreduce(accs, jnp.add)
        conf16 = jnp.exp(m16) / s16

        r16 = jnp.full((16,), r, jnp.int32)
        lab16 = plsc.load_gather(lab_ref, [r16])
        labval16 = plsc.load_gather(buf, [base + lab16])
        accf16 = jnp.where(labval16 == m16, 1.0, 0.0)

        inb = jnp.logical_and(conf16 > lo16, conf16 <= hi16)
        mask = jnp.where(inb, 1.0, 0.0)
        return (cnt + mask, csum + mask * conf16, asum + mask * accf16)

    return body


def _sc_body(logits_ref, labels_ref, bounds_ref, out_ref,
             buf0, buf1, lab0, lab1, cb, stage, rb,
             semA0, semA1, semB0, semB1):
    wid = lax.axis_index("s") * 2 + lax.axis_index("c")
    base_row = wid * ROWS_PER_TILE
    lane = lax.iota(jnp.int32, 16)

    pltpu.sync_copy(bounds_ref, cb)
    lo16 = cb[0]
    hi16 = cb[1]

    def data_cp(c, bf, sem):
        start = (base_row + c * CH) * N_COLS
        return pltpu.make_async_copy(
            logits_ref.at[pl.ds(start, CH * N_COLS)], bf, sem)

    def lab_cp(c, lb, sem):
        start = base_row + c * CH
        return pltpu.make_async_copy(
            labels_ref.at[pl.ds(start, CH)], lb, sem)

    data_cp(0, buf0, semA0).start()
    lab_cp(0, lab0, semB0).start()

    bufs = ((buf0, lab0, semA0, semB0), (buf1, lab1, semA1, semB1))

    def pair_body(cp, carry):
        for b in range(2):
            bf, lb, sA, sB = bufs[b]
            obf, olb, osA, osB = bufs[1 - b]
            c = 2 * cp + b
            data_cp(c, bf, sA).wait()
            lab_cp(c, lb, sB).wait()

            @pl.when(c + 1 < NCHUNKS)
            def _prefetch():
                data_cp(c + 1, obf, osA).start()
                lab_cp(c + 1, olb, osB).start()

            carry = lax.fori_loop(
                0, CH, _row_body(bf, lb, rb, lo16, hi16, lane), carry)
        return carry

    zeros = jnp.zeros((16,), jnp.float32)
    cnt, csum, asum = lax.fori_loop(
        0, NCHUNKS // 2, pair_body, (zeros, zeros, zeros))

    stage[0, :] = cnt
    stage[1, :] = csum
    stage[2, :] = asum
    pltpu.sync_copy(stage, out_ref.at[wid])


def _ece_combine(parts_ref, out_ref, *, n_total):
    p = jnp.sum(parts_ref[...], axis=0)
    cnt = p[0]
    csum = p[1]
    asum = p[2]
    prop = cnt / n_total
    denom = jnp.maximum(cnt, 1.0)
    contrib = jnp.abs(csum / denom - asum / denom) * prop
    out_ref[0, 0] = jnp.sum(jnp.where(cnt > 0.0, contrib, 0.0))


@jax.jit
def kernel(logits, labels):
    bounds = jnp.linspace(0.0, 1.0, N_BINS + 1).astype(jnp.float32)
    lo = jnp.concatenate([bounds[:N_BINS], jnp.ones((1,), jnp.float32)])
    hi = jnp.concatenate([bounds[1:], jnp.ones((1,), jnp.float32)])
    bounds2 = jnp.stack([lo, hi])

    sc = pl.kernel(
        _sc_body,
        out_type=jax.ShapeDtypeStruct((NTILES, 3, 16), jnp.float32),
        mesh=plsc.VectorSubcoreMesh(core_axis_name="c", subcore_axis_name="s"),
        compiler_params=pltpu.CompilerParams(needs_layout_passes=False),
        scratch_types=[
            pltpu.VMEM((CH * N_COLS,), jnp.float32),
            pltpu.VMEM((CH * N_COLS,), jnp.float32),
            pltpu.VMEM((CH,), jnp.int32),
            pltpu.VMEM((CH,), jnp.int32),
            pltpu.VMEM((2, 16), jnp.float32),
            pltpu.VMEM((3, 16), jnp.float32),
            pltpu.VMEM((16,), jnp.float32),
            pltpu.SemaphoreType.DMA,
            pltpu.SemaphoreType.DMA,
            pltpu.SemaphoreType.DMA,
            pltpu.SemaphoreType.DMA,
        ],
    )
    parts = sc(logits.reshape(-1), labels, bounds2)

    out = pl.pallas_call(
        functools.partial(_ece_combine, n_total=float(N_ROWS)),
        out_specs=pl.BlockSpec(memory_space=pltpu.SMEM),
        out_shape=jax.ShapeDtypeStruct((1, 1), jnp.float32),
    )(parts)
    return out[0, 0]


# speedup vs baseline: 1.0218x; 1.0218x over previous
import functools

import jax
import jax.numpy as jnp
from jax import lax
from jax.experimental import pallas as pl
from jax.experimental.pallas import tpu as pltpu
from jax.experimental.pallas import tpu_sc as plsc

N_BINS = 15
N_ROWS = 16384
N_COLS = 1000
NTILES = 32
N_SC = 6144
N_TC = N_ROWS - N_SC
ROWS_PER_TILE = N_SC // NTILES
CH = 32
NCHUNKS = ROWS_PER_TILE // CH
NVREG = N_COLS // 16
TAIL = N_COLS - NVREG * 16
NEG = -1e30
TC_BLOCK = 1024
TC_STEPS = N_TC // TC_BLOCK


def _row_body(buf, lab_ref, rb, lo16, hi16, lane):
    def allreduce(v, op):
        for k in (8, 4, 2, 1):
            rb[...] = v
            vs = plsc.load_gather(rb, [jnp.bitwise_xor(lane, k)])
            v = op(v, vs)
        return v

    def body(r, carry):
        cnt, csum, asum = carry
        base = r * N_COLS
        accm = jnp.full((16,), NEG, jnp.float32)
        accs = jnp.zeros((16,), jnp.float32)
        for i in range(NVREG):
            v = buf[pl.ds(base + i * 16, 16)]
            accm = jnp.maximum(accm, v)
            accs = accs + jnp.exp(v)
        vt = buf[pl.ds(base + N_COLS - 16, 16)]
        sel = lane >= (16 - TAIL)
        accm = jnp.maximum(accm, jnp.where(sel, vt, NEG))
        accs = accs + jnp.where(sel, jnp.exp(vt), 0.0)

        m16 = allreduce(accm, jnp.maximum)
        s16 = allreduce(accs, jnp.add)
        conf16 = jnp.exp(m16) / s16

        r16 = jnp.full((16,), r, jnp.int32)
        lab16 = plsc.load_gather(lab_ref, [r16])
        labval16 = plsc.load_gather(buf, [base + lab16])
        accf16 = jnp.where(labval16 == m16, 1.0, 0.0)

        inb = jnp.logical_and(conf16 > lo16, conf16 <= hi16)
        mask = jnp.where(inb, 1.0, 0.0)
        return (cnt + mask, csum + mask * conf16, asum + mask * accf16)

    return body


def _sc_body(logits_ref, labels_ref, bounds_ref, out_ref,
             buf0, buf1, lab0, lab1, cb, stage, rb,
             semA0, semA1, semB0, semB1):
    wid = lax.axis_index("s") * 2 + lax.axis_index("c")
    base_row = wid * ROWS_PER_TILE
    lane = lax.iota(jnp.int32, 16)

    pltpu.sync_copy(bounds_ref, cb)
    lo16 = cb[0]
    hi16 = cb[1]

    def data_cp(c, bf, sem):
        start = (base_row + c * CH) * N_COLS
        return pltpu.make_async_copy(
            logits_ref.at[pl.ds(start, CH * N_COLS)], bf, sem)

    def lab_cp(c, lb, sem):
        start = base_row + c * CH
        return pltpu.make_async_copy(
            labels_ref.at[pl.ds(start, CH)], lb, sem)

    data_cp(0, buf0, semA0).start()
    lab_cp(0, lab0, semB0).start()

    bufs = ((buf0, lab0, semA0, semB0), (buf1, lab1, semA1, semB1))

    def pair_body(cp, carry):
        for b in range(2):
            bf, lb, sA, sB = bufs[b]
            obf, olb, osA, osB = bufs[1 - b]
            c = 2 * cp + b
            data_cp(c, bf, sA).wait()
            lab_cp(c, lb, sB).wait()

            @pl.when(c + 1 < NCHUNKS)
            def _prefetch():
                data_cp(c + 1, obf, osA).start()
                lab_cp(c + 1, olb, osB).start()

            carry = lax.fori_loop(
                0, CH, _row_body(bf, lb, rb, lo16, hi16, lane), carry)
        return carry

    zeros = jnp.zeros((16,), jnp.float32)
    cnt, csum, asum = lax.fori_loop(
        0, NCHUNKS // 2, pair_body, (zeros, zeros, zeros))

    stage[0, :] = cnt
    stage[1, :] = csum
    stage[2, :] = asum
    pltpu.sync_copy(stage, out_ref.at[wid])


def _tc_main(logits_ref, labels_ref, lo_ref, hi_ref, part_ref,
             cnt_acc, conf_acc, acc_acc):
    j = pl.program_id(0)

    x = logits_ref[...]
    m = jnp.max(x, axis=1, keepdims=True)
    s = jnp.sum(jnp.exp(x - m), axis=1, keepdims=True)
    conf = 1.0 / s

    col = lax.broadcasted_iota(jnp.int32, x.shape, 1)
    idx = jnp.min(jnp.where(x == m, col, N_COLS), axis=1,
                  keepdims=True)
    acc = (idx == labels_ref[0]).astype(jnp.float32)

    mask = jnp.logical_and(conf > lo_ref[...], conf <= hi_ref[...])
    mask = mask.astype(jnp.float32)

    @pl.when(j == 0)
    def _init():
        cnt_acc[...] = jnp.zeros_like(cnt_acc)
        conf_acc[...] = jnp.zeros_like(conf_acc)
        acc_acc[...] = jnp.zeros_like(acc_acc)

    cnt_acc[...] += mask
    conf_acc[...] += mask * conf
    acc_acc[...] += mask * acc

    @pl.when(j == TC_STEPS - 1)
    def _finish():
        cnt = jnp.sum(cnt_acc[...], axis=0, keepdims=True)
        csum = jnp.sum(conf_acc[...], axis=0, keepdims=True)
        asum = jnp.sum(acc_acc[...], axis=0, keepdims=True)
        pad = jnp.zeros((8 - 3, 16), jnp.float32)
        part_ref[...] = jnp.concatenate([cnt, csum, asum, pad], axis=0)


def _ece_combine(sc_ref, tc_ref, out_ref, *, n_total):
    p = jnp.sum(sc_ref[...], axis=0)
    t = tc_ref[...]
    cnt = p[0] + t[0]
    csum = p[1] + t[1]
    asum = p[2] + t[2]
    prop = cnt / n_total
    denom = jnp.maximum(cnt, 1.0)
    contrib = jnp.abs(csum / denom - asum / denom) * prop
    out_ref[0, 0] = jnp.sum(jnp.where(cnt > 0.0, contrib, 0.0))


@jax.jit
def kernel(logits, labels):
    bounds = jnp.linspace(0.0, 1.0, N_BINS + 1).astype(jnp.float32)
    lo = jnp.concatenate([bounds[:N_BINS], jnp.ones((1,), jnp.float32)])
    hi = jnp.concatenate([bounds[1:], jnp.ones((1,), jnp.float32)])
    bounds2 = jnp.stack([lo, hi])

    sc = pl.kernel(
        _sc_body,
        out_type=jax.ShapeDtypeStruct((NTILES, 3, 16), jnp.float32),
        mesh=plsc.VectorSubcoreMesh(core_axis_name="c", subcore_axis_name="s"),
        compiler_params=pltpu.CompilerParams(needs_layout_passes=False),
        scratch_types=[
            pltpu.VMEM((CH * N_COLS,), jnp.float32),
            pltpu.VMEM((CH * N_COLS,), jnp.float32),
            pltpu.VMEM((CH,), jnp.int32),
            pltpu.VMEM((CH,), jnp.int32),
            pltpu.VMEM((2, 16), jnp.float32),
            pltpu.VMEM((3, 16), jnp.float32),
            pltpu.VMEM((16,), jnp.float32),
            pltpu.SemaphoreType.DMA,
            pltpu.SemaphoreType.DMA,
            pltpu.SemaphoreType.DMA,
            pltpu.SemaphoreType.DMA,
        ],
    )
    sc_parts = sc(logits[:N_SC].reshape(-1), labels[:N_SC], bounds2)

    labels_tc = labels[N_SC:].reshape(TC_STEPS, TC_BLOCK, 1)
    tc_parts = pl.pallas_call(
        _tc_main,
        grid=(TC_STEPS,),
        in_specs=[
            pl.BlockSpec((TC_BLOCK, N_COLS), lambda j: (j, 0)),
            pl.BlockSpec((1, TC_BLOCK, 1), lambda j: (j, 0, 0)),
            pl.BlockSpec((1, 16), lambda j: (0, 0)),
            pl.BlockSpec((1, 16), lambda j: (0, 0)),
        ],
        out_specs=pl.BlockSpec((8, 16), lambda j: (0, 0)),
        out_shape=jax.ShapeDtypeStruct((8, 16), jnp.float32),
        scratch_shapes=[
            pltpu.VMEM((TC_BLOCK, 16), jnp.float32),
            pltpu.VMEM((TC_BLOCK, 16), jnp.float32),
            pltpu.VMEM((TC_BLOCK, 16), jnp.float32),
        ],
        compiler_params=pltpu.CompilerParams(
            dimension_semantics=("arbitrary",)),
    )(logits[N_SC:], labels_tc, lo.reshape(1, 16), hi.reshape(1, 16))

    out = pl.pallas_call(
        functools.partial(_ece_combine, n_total=float(N_ROWS)),
        out_specs=pl.BlockSpec(memory_space=pltpu.SMEM),
        out_shape=jax.ShapeDtypeStruct((1, 1), jnp.float32),
    )(sc_parts, tc_parts)
    return out[0, 0]


# speedup vs baseline: 1.0744x; 1.0515x over previous
import functools

import jax
import jax.numpy as jnp
from jax import lax
from jax.experimental import pallas as pl
from jax.experimental.pallas import tpu as pltpu
from jax.experimental.pallas import tpu_sc as plsc

N_BINS = 15
N_ROWS = 16384
N_COLS = 1000
NTILES = 32
N_SC = 8192
N_TC = N_ROWS - N_SC
ROWS_PER_TILE = N_SC // NTILES
CH = 32
NCHUNKS = ROWS_PER_TILE // CH
NVREG = N_COLS // 16
TAIL = N_COLS - NVREG * 16
NEG = -1e30
TC_BLOCK = 1024
TC_STEPS = N_TC // TC_BLOCK


def _row_body(buf, lab_ref, rb, lo16, hi16, lane):
    def allreduce(v, op):
        for k in (8, 4, 2, 1):
            rb[...] = v
            vs = plsc.load_gather(rb, [jnp.bitwise_xor(lane, k)])
            v = op(v, vs)
        return v

    def body(r, carry):
        cnt, csum, asum = carry
        base = r * N_COLS
        accm = jnp.full((16,), NEG, jnp.float32)
        accs = jnp.zeros((16,), jnp.float32)
        for i in range(NVREG):
            v = buf[pl.ds(base + i * 16, 16)]
            accm = jnp.maximum(accm, v)
            accs = accs + jnp.exp(v)
        vt = buf[pl.ds(base + N_COLS - 16, 16)]
        sel = lane >= (16 - TAIL)
        accm = jnp.maximum(accm, jnp.where(sel, vt, NEG))
        accs = accs + jnp.where(sel, jnp.exp(vt), 0.0)

        m16 = allreduce(accm, jnp.maximum)
        s16 = allreduce(accs, jnp.add)
        conf16 = jnp.exp(m16) / s16

        r16 = jnp.full((16,), r, jnp.int32)
        lab16 = plsc.load_gather(lab_ref, [r16])
        labval16 = plsc.load_gather(buf, [base + lab16])
        accf16 = jnp.where(labval16 == m16, 1.0, 0.0)

        inb = jnp.logical_and(conf16 > lo16, conf16 <= hi16)
        mask = jnp.where(inb, 1.0, 0.0)
        return (cnt + mask, csum + mask * conf16, asum + mask * accf16)

    return body


def _sc_body(logits_ref, labels_ref, bounds_ref, out_ref,
             buf0, buf1, lab0, lab1, cb, stage, rb,
             semA0, semA1, semB0, semB1):
    wid = lax.axis_index("s") * 2 + lax.axis_index("c")
    base_row = wid * ROWS_PER_TILE
    lane = lax.iota(jnp.int32, 16)

    pltpu.sync_copy(bounds_ref, cb)
    lo16 = cb[0]
    hi16 = cb[1]

    def data_cp(c, bf, sem):
        start = (base_row + c * CH) * N_COLS
        return pltpu.make_async_copy(
            logits_ref.at[pl.ds(start, CH * N_COLS)], bf, sem)

    def lab_cp(c, lb, sem):
        start = base_row + c * CH
        return pltpu.make_async_copy(
            labels_ref.at[pl.ds(start, CH)], lb, sem)

    data_cp(0, buf0, semA0).start()
    lab_cp(0, lab0, semB0).start()

    bufs = ((buf0, lab0, semA0, semB0), (buf1, lab1, semA1, semB1))

    def pair_body(cp, carry):
        for b in range(2):
            bf, lb, sA, sB = bufs[b]
            obf, olb, osA, osB = bufs[1 - b]
            c = 2 * cp + b
            data_cp(c, bf, sA).wait()
            lab_cp(c, lb, sB).wait()

            @pl.when(c + 1 < NCHUNKS)
            def _prefetch():
                data_cp(c + 1, obf, osA).start()
                lab_cp(c + 1, olb, osB).start()

            carry = lax.fori_loop(
                0, CH, _row_body(bf, lb, rb, lo16, hi16, lane), carry)
        return carry

    zeros = jnp.zeros((16,), jnp.float32)
    cnt, csum, asum = lax.fori_loop(
        0, NCHUNKS // 2, pair_body, (zeros, zeros, zeros))

    stage[0, :] = cnt
    stage[1, :] = csum
    stage[2, :] = asum
    pltpu.sync_copy(stage, out_ref.at[wid])


def _tc_main(logits_ref, labels_ref, lo_ref, hi_ref, part_ref,
             cnt_acc, conf_acc, acc_acc):
    j = pl.program_id(0)

    x = logits_ref[...]
    m = jnp.max(x, axis=1, keepdims=True)
    s = jnp.sum(jnp.exp(x - m), axis=1, keepdims=True)
    conf = 1.0 / s

    col = lax.broadcasted_iota(jnp.int32, x.shape, 1)
    idx = jnp.min(jnp.where(x == m, col, N_COLS), axis=1,
                  keepdims=True)
    acc = (idx == labels_ref[0]).astype(jnp.float32)

    mask = jnp.logical_and(conf > lo_ref[...], conf <= hi_ref[...])
    mask = mask.astype(jnp.float32)

    @pl.when(j == 0)
    def _init():
        cnt_acc[...] = jnp.zeros_like(cnt_acc)
        conf_acc[...] = jnp.zeros_like(conf_acc)
        acc_acc[...] = jnp.zeros_like(acc_acc)

    cnt_acc[...] += mask
    conf_acc[...] += mask * conf
    acc_acc[...] += mask * acc

    @pl.when(j == TC_STEPS - 1)
    def _finish():
        cnt = jnp.sum(cnt_acc[...], axis=0, keepdims=True)
        csum = jnp.sum(conf_acc[...], axis=0, keepdims=True)
        asum = jnp.sum(acc_acc[...], axis=0, keepdims=True)
        pad = jnp.zeros((8 - 3, 16), jnp.float32)
        part_ref[...] = jnp.concatenate([cnt, csum, asum, pad], axis=0)


def _ece_combine(sc_ref, tc_ref, out_ref, *, n_total):
    p = jnp.sum(sc_ref[...], axis=0)
    t = tc_ref[...]
    cnt = p[0] + t[0]
    csum = p[1] + t[1]
    asum = p[2] + t[2]
    prop = cnt / n_total
    denom = jnp.maximum(cnt, 1.0)
    contrib = jnp.abs(csum / denom - asum / denom) * prop
    out_ref[0, 0] = jnp.sum(jnp.where(cnt > 0.0, contrib, 0.0))


@jax.jit
def kernel(logits, labels):
    bounds = jnp.linspace(0.0, 1.0, N_BINS + 1).astype(jnp.float32)
    lo = jnp.concatenate([bounds[:N_BINS], jnp.ones((1,), jnp.float32)])
    hi = jnp.concatenate([bounds[1:], jnp.ones((1,), jnp.float32)])
    bounds2 = jnp.stack([lo, hi])

    sc = pl.kernel(
        _sc_body,
        out_type=jax.ShapeDtypeStruct((NTILES, 3, 16), jnp.float32),
        mesh=plsc.VectorSubcoreMesh(core_axis_name="c", subcore_axis_name="s"),
        compiler_params=pltpu.CompilerParams(needs_layout_passes=False),
        scratch_types=[
            pltpu.VMEM((CH * N_COLS,), jnp.float32),
            pltpu.VMEM((CH * N_COLS,), jnp.float32),
            pltpu.VMEM((CH,), jnp.int32),
            pltpu.VMEM((CH,), jnp.int32),
            pltpu.VMEM((2, 16), jnp.float32),
            pltpu.VMEM((3, 16), jnp.float32),
            pltpu.VMEM((16,), jnp.float32),
            pltpu.SemaphoreType.DMA,
            pltpu.SemaphoreType.DMA,
            pltpu.SemaphoreType.DMA,
            pltpu.SemaphoreType.DMA,
        ],
    )
    sc_parts = sc(logits.reshape(-1), labels, bounds2)

    tc_off = N_SC // TC_BLOCK
    labels3d = labels.reshape(N_ROWS // TC_BLOCK, TC_BLOCK, 1)
    tc_parts = pl.pallas_call(
        _tc_main,
        grid=(TC_STEPS,),
        in_specs=[
            pl.BlockSpec((TC_BLOCK, N_COLS), lambda j: (j + tc_off, 0)),
            pl.BlockSpec((1, TC_BLOCK, 1), lambda j: (j + tc_off, 0, 0)),
            pl.BlockSpec((1, 16), lambda j: (0, 0)),
            pl.BlockSpec((1, 16), lambda j: (0, 0)),
        ],
        out_specs=pl.BlockSpec((8, 16), lambda j: (0, 0)),
        out_shape=jax.ShapeDtypeStruct((8, 16), jnp.float32),
        scratch_shapes=[
            pltpu.VMEM((TC_BLOCK, 16), jnp.float32),
            pltpu.VMEM((TC_BLOCK, 16), jnp.float32),
            pltpu.VMEM((TC_BLOCK, 16), jnp.float32),
        ],
        compiler_params=pltpu.CompilerParams(
            dimension_semantics=("arbitrary",)),
    )(logits, labels3d, lo.reshape(1, 16), hi.reshape(1, 16))

    out = pl.pallas_call(
        functools.partial(_ece_combine, n_total=float(N_ROWS)),
        out_specs=pl.BlockSpec(memory_space=pltpu.SMEM),
        out_shape=jax.ShapeDtypeStruct((1, 1), jnp.float32),
    )(sc_parts, tc_parts)
    return out[0, 0]


# speedup vs baseline: 1.8639x; 1.7348x over previous
import functools

import jax
import jax.numpy as jnp
from jax import lax
from jax.experimental import pallas as pl
from jax.experimental.pallas import tpu as pltpu
from jax.experimental.pallas import tpu_sc as plsc

N_BINS = 15
N_ROWS = 16384
N_COLS = 1000
NTILES = 32
N_SC = 8192
N_TC = N_ROWS - N_SC
ROWS_PER_TILE = N_SC // NTILES
CH = 32
NCHUNKS = ROWS_PER_TILE // CH
NVREG = N_COLS // 16
TAIL = N_COLS - NVREG * 16
NEG = -1e30
TC_BLOCK = 1024
TC_STEPS = N_TC // TC_BLOCK


def _row_body(buf, lab_ref, rb, lo16, hi16, lane):
    def allreduce(v, op):
        for k in (8, 4, 2, 1):
            rb[...] = v
            vs = plsc.load_gather(rb, [jnp.bitwise_xor(lane, k)])
            v = op(v, vs)
        return v

    def body(r, carry):
        cnt, csum, asum = carry
        accm = jnp.full((16,), NEG, jnp.float32)
        accs = jnp.zeros((16,), jnp.float32)
        for i in range(NVREG):
            v = buf[r, pl.ds(i * 16, 16)]
            accm = jnp.maximum(accm, v)
            accs = accs + jnp.exp(v)
        vt = buf[r, pl.ds(N_COLS - 16, 16)]
        sel = lane >= (16 - TAIL)
        accm = jnp.maximum(accm, jnp.where(sel, vt, NEG))
        accs = accs + jnp.where(sel, jnp.exp(vt), 0.0)

        m16 = allreduce(accm, jnp.maximum)
        s16 = allreduce(accs, jnp.add)
        conf16 = jnp.exp(m16) / s16

        r16 = jnp.full((16,), r, jnp.int32)
        lab16 = plsc.load_gather(lab_ref, [r16])
        labval16 = plsc.load_gather(buf, [r16, lab16])
        accf16 = jnp.where(labval16 == m16, 1.0, 0.0)

        inb = jnp.logical_and(conf16 > lo16, conf16 <= hi16)
        mask = jnp.where(inb, 1.0, 0.0)
        return (cnt + mask, csum + mask * conf16, asum + mask * accf16)

    return body


def _sc_body(logits_ref, labels_ref, bounds_ref, out_ref,
             buf0, buf1, lab0, lab1, cb, stage, rb,
             semA0, semA1, semB0, semB1):
    wid = lax.axis_index("s") * 2 + lax.axis_index("c")
    base_row = wid * ROWS_PER_TILE
    lane = lax.iota(jnp.int32, 16)

    pltpu.sync_copy(bounds_ref, cb)
    lo16 = cb[0]
    hi16 = cb[1]

    def data_cp(c, bf, sem):
        start = base_row + c * CH
        return pltpu.make_async_copy(
            logits_ref.at[pl.ds(start, CH)], bf, sem)

    def lab_cp(c, lb, sem):
        start = base_row + c * CH
        return pltpu.make_async_copy(
            labels_ref.at[pl.ds(start, CH)], lb, sem)

    data_cp(0, buf0, semA0).start()
    lab_cp(0, lab0, semB0).start()

    bufs = ((buf0, lab0, semA0, semB0), (buf1, lab1, semA1, semB1))

    def pair_body(cp, carry):
        for b in range(2):
            bf, lb, sA, sB = bufs[b]
            obf, olb, osA, osB = bufs[1 - b]
            c = 2 * cp + b
            data_cp(c, bf, sA).wait()
            lab_cp(c, lb, sB).wait()

            @pl.when(c + 1 < NCHUNKS)
            def _prefetch():
                data_cp(c + 1, obf, osA).start()
                lab_cp(c + 1, olb, osB).start()

            carry = lax.fori_loop(
                0, CH, _row_body(bf, lb, rb, lo16, hi16, lane), carry)
        return carry

    zeros = jnp.zeros((16,), jnp.float32)
    cnt, csum, asum = lax.fori_loop(
        0, NCHUNKS // 2, pair_body, (zeros, zeros, zeros))

    stage[0, :] = cnt
    stage[1, :] = csum
    stage[2, :] = asum
    pltpu.sync_copy(stage, out_ref.at[wid])


def _tc_main(logits_ref, labels_ref, lo_ref, hi_ref, part_ref,
             cnt_acc, conf_acc, acc_acc):
    j = pl.program_id(0)

    x = logits_ref[...]
    m = jnp.max(x, axis=1, keepdims=True)
    s = jnp.sum(jnp.exp(x - m), axis=1, keepdims=True)
    conf = 1.0 / s

    col = lax.broadcasted_iota(jnp.int32, x.shape, 1)
    idx = jnp.min(jnp.where(x == m, col, N_COLS), axis=1,
                  keepdims=True)
    acc = (idx == labels_ref[0]).astype(jnp.float32)

    mask = jnp.logical_and(conf > lo_ref[...], conf <= hi_ref[...])
    mask = mask.astype(jnp.float32)

    @pl.when(j == 0)
    def _init():
        cnt_acc[...] = jnp.zeros_like(cnt_acc)
        conf_acc[...] = jnp.zeros_like(conf_acc)
        acc_acc[...] = jnp.zeros_like(acc_acc)

    cnt_acc[...] += mask
    conf_acc[...] += mask * conf
    acc_acc[...] += mask * acc

    @pl.when(j == TC_STEPS - 1)
    def _finish():
        cnt = jnp.sum(cnt_acc[...], axis=0, keepdims=True)
        csum = jnp.sum(conf_acc[...], axis=0, keepdims=True)
        asum = jnp.sum(acc_acc[...], axis=0, keepdims=True)
        pad = jnp.zeros((8 - 3, 16), jnp.float32)
        part_ref[...] = jnp.concatenate([cnt, csum, asum, pad], axis=0)


def _ece_combine(sc_ref, tc_ref, out_ref, *, n_total):
    p = jnp.sum(sc_ref[...], axis=0)
    t = tc_ref[...]
    cnt = p[0] + t[0]
    csum = p[1] + t[1]
    asum = p[2] + t[2]
    prop = cnt / n_total
    denom = jnp.maximum(cnt, 1.0)
    contrib = jnp.abs(csum / denom - asum / denom) * prop
    out_ref[0, 0] = jnp.sum(jnp.where(cnt > 0.0, contrib, 0.0))


@jax.jit
def kernel(logits, labels):
    bounds = jnp.linspace(0.0, 1.0, N_BINS + 1).astype(jnp.float32)
    lo = jnp.concatenate([bounds[:N_BINS], jnp.ones((1,), jnp.float32)])
    hi = jnp.concatenate([bounds[1:], jnp.ones((1,), jnp.float32)])
    bounds2 = jnp.stack([lo, hi])

    sc = pl.kernel(
        _sc_body,
        out_type=jax.ShapeDtypeStruct((NTILES, 3, 16), jnp.float32),
        mesh=plsc.VectorSubcoreMesh(core_axis_name="c", subcore_axis_name="s"),
        compiler_params=pltpu.CompilerParams(needs_layout_passes=False),
        scratch_types=[
            pltpu.VMEM((CH, N_COLS), jnp.float32),
            pltpu.VMEM((CH, N_COLS), jnp.float32),
            pltpu.VMEM((CH,), jnp.int32),
            pltpu.VMEM((CH,), jnp.int32),
            pltpu.VMEM((2, 16), jnp.float32),
            pltpu.VMEM((3, 16), jnp.float32),
            pltpu.VMEM((16,), jnp.float32),
            pltpu.SemaphoreType.DMA,
            pltpu.SemaphoreType.DMA,
            pltpu.SemaphoreType.DMA,
            pltpu.SemaphoreType.DMA,
        ],
    )
    sc_parts = sc(logits, labels, bounds2)

    tc_off = N_SC // TC_BLOCK
    labels3d = labels.reshape(N_ROWS // TC_BLOCK, TC_BLOCK, 1)
    tc_parts = pl.pallas_call(
        _tc_main,
        grid=(TC_STEPS,),
        in_specs=[
            pl.BlockSpec((TC_BLOCK, N_COLS), lambda j: (j + tc_off, 0)),
            pl.BlockSpec((1, TC_BLOCK, 1), lambda j: (j + tc_off, 0, 0)),
            pl.BlockSpec((1, 16), lambda j: (0, 0)),
            pl.BlockSpec((1, 16), lambda j: (0, 0)),
        ],
        out_specs=pl.BlockSpec((8, 16), lambda j: (0, 0)),
        out_shape=jax.ShapeDtypeStruct((8, 16), jnp.float32),
        scratch_shapes=[
            pltpu.VMEM((TC_BLOCK, 16), jnp.float32),
            pltpu.VMEM((TC_BLOCK, 16), jnp.float32),
            pltpu.VMEM((TC_BLOCK, 16), jnp.float32),
        ],
        compiler_params=pltpu.CompilerParams(
            dimension_semantics=("arbitrary",)),
    )(logits, labels3d, lo.reshape(1, 16), hi.reshape(1, 16))

    out = pl.pallas_call(
        functools.partial(_ece_combine, n_total=float(N_ROWS)),
        out_specs=pl.BlockSpec(memory_space=pltpu.SMEM),
        out_shape=jax.ShapeDtypeStruct((1, 1), jnp.float32),
    )(sc_parts, tc_parts)
    return out[0, 0]
